# baseline (speedup 1.0000x reference)
import functools

import jax
import jax.numpy as jnp
from jax.experimental import pallas as pl
from jax.experimental.pallas import tpu as pltpu

K_NEG = 25
K_NEIGH = K_NEG * 40
POS_MARGIN = 0.01
NEG_MARGIN = 2.0
NEG_PARAM = 0.2

N = 8192
D = 256
BR = 512
BC = 2048


def _sim_body(a_ref, b_ref, an_ref, bn_ref, o_ref):
    a = a_ref[...]
    b = b_ref[...]
    an = an_ref[...]
    bn = bn_ref[...]
    dot = jax.lax.dot_general(a, b, (((1,), (1,)), ((), ())))
    d = an.T + bn - 2.0 * dot
    o_ref[...] = jnp.maximum(d, 0.0)


def kernel(entity_emb, rel_emb, pos_triples, neg_triples):
    en = jnp.sum(entity_emb * entity_emb, axis=1, keepdims=True)
    sim = pl.pallas_call(
        _sim_body,
        grid=(N // BR, N // BC),
        in_specs=[
            pl.BlockSpec((BR, D), lambda i, j: (i, 0)),
            pl.BlockSpec((BC, D), lambda i, j: (j, 0)),
            pl.BlockSpec((1, BR), lambda i, j: (0, i)),
            pl.BlockSpec((1, BC), lambda i, j: (0, j)),
        ],
        out_specs=pl.BlockSpec((BR, BC), lambda i, j: (i, j)),
        out_shape=jax.ShapeDtypeStruct((N, N), jnp.float32),
    )(entity_emb, entity_emb, en.T, en.T)

    _, nn_idx = jax.lax.top_k(-sim, K_NEIGH + 1)
    neighbours = nn_idx[:, 1:K_NEIGH + 1]

    pos_head = jnp.take(entity_emb, pos_triples[:, 0], axis=0)
    pos_rel = jnp.take(rel_emb, pos_triples[:, 1], axis=0)
    pos_tail = jnp.take(entity_emb, pos_triples[:, 2], axis=0)
    neg_head = jnp.take(entity_emb, neg_triples[:, 0], axis=0)
    neg_rel = jnp.take(rel_emb, neg_triples[:, 1], axis=0)
    neg_tail = jnp.take(entity_emb, neg_triples[:, 2], axis=0)

    pos_score = jnp.sum((pos_head + pos_rel - pos_tail) ** 2, axis=1)
    neg_score = jnp.sum((neg_head + neg_rel - neg_tail) ** 2, axis=1)

    pos_loss = jax.nn.relu(pos_score - POS_MARGIN).sum()
    neg_loss = jax.nn.relu(NEG_MARGIN - neg_score).sum()
    loss = pos_loss + NEG_PARAM * neg_loss
    return loss, neighbours
